# baseline (speedup 1.0000x reference)
import functools

import jax
import jax.numpy as jnp
from jax import lax
from jax.experimental import pallas as pl
from jax.experimental.pallas import tpu as pltpu
from jax.experimental.pallas import tpu_sc as plsc

SEQ = 200
BATCH = 4096
EMBED = 64
OUT = 32


def _make_sc_pool(num_cores, num_subcores):
    nw = num_cores * num_subcores
    b_per_w = BATCH // nw
    mesh = plsc.VectorSubcoreMesh(
        core_axis_name="c", subcore_axis_name="s",
        num_cores=num_cores, num_subcores=num_subcores)

    @functools.partial(
        pl.kernel,
        mesh=mesh,
        out_type=jax.ShapeDtypeStruct((BATCH, EMBED), jnp.float32),
        scratch_types=[
            pltpu.VMEM((SEQ, b_per_w), jnp.int32),
            pltpu.VMEM((b_per_w, EMBED), jnp.float32),
            pltpu.VMEM((b_per_w, EMBED), jnp.float32),
            pltpu.VMEM((b_per_w, EMBED), jnp.float32),
            pltpu.SemaphoreType.DMA,
            pltpu.SemaphoreType.DMA,
        ],
        compiler_params=pltpu.CompilerParams(use_tc_tiling_on_sc=False),
    )
    def sc_pool(x_hbm, table_hbm, out_hbm, idx_v, rows_a, rows_b, acc_v,
                sem_a, sem_b):
        wid = lax.axis_index("s") * num_cores + lax.axis_index("c")
        base = wid * b_per_w

        pltpu.sync_copy(x_hbm.at[:, pl.ds(base, b_per_w)], idx_v)

        z = jnp.zeros((16,), jnp.float32)

        def zero_body(b, _):
            for j in range(EMBED // 16):
                acc_v[b, pl.ds(16 * j, 16)] = z
            return 0

        lax.fori_loop(0, b_per_w, zero_body, 0)

        def gather(s, rows, sem):
            pltpu.async_copy(table_hbm.at[idx_v.at[s]], rows, sem)

        def wait(rows, sem):
            pltpu.make_async_copy(table_hbm.at[pl.ds(0, b_per_w)], rows,
                                  sem).wait()

        def accum(rows):
            u = 4

            def body(i, _):
                b = i * u
                for k in range(u):
                    for j in range(EMBED // 16):
                        plsc.addupdate(acc_v.at[b + k, pl.ds(16 * j, 16)],
                                       rows[b + k, pl.ds(16 * j, 16)])
                return 0

            lax.fori_loop(0, b_per_w // u, body, 0)

        gather(0, rows_a, sem_a)
        gather(1, rows_b, sem_b)

        def pair(i, _):
            s0 = 2 * i
            wait(rows_a, sem_a)

            @pl.when(s0 + 2 < SEQ)
            def _():
                gather(s0 + 2, rows_a, sem_a)

            accum(rows_a)
            wait(rows_b, sem_b)

            @pl.when(s0 + 3 < SEQ)
            def _():
                gather(s0 + 3, rows_b, sem_b)

            accum(rows_b)
            return 0

        lax.fori_loop(0, SEQ // 2, pair, 0)

        pltpu.sync_copy(acc_v, out_hbm.at[pl.ds(base, b_per_w)])

    return sc_pool


def _linear_body(p_ref, w_ref, b_ref, o_ref):
    p = p_ref[...]
    w = w_ref[...]
    acc = lax.dot_general(p, w, (((1,), (1,)), ((), ())),
                          preferred_element_type=jnp.float32)
    o_ref[...] = acc * (1.0 / SEQ) + b_ref[...]


def _linear(pooled_sum, fc_w, fc_b2):
    blk = 512
    return pl.pallas_call(
        _linear_body,
        grid=(BATCH // blk,),
        in_specs=[
            pl.BlockSpec((blk, EMBED), lambda i: (i, 0)),
            pl.BlockSpec((OUT, EMBED), lambda i: (0, 0)),
            pl.BlockSpec((1, OUT), lambda i: (0, 0)),
        ],
        out_specs=pl.BlockSpec((blk, OUT), lambda i: (i, 0)),
        out_shape=jax.ShapeDtypeStruct((BATCH, OUT), jnp.float32),
    )(pooled_sum, fc_w, fc_b2)


def kernel(x, emb_table, fc_w, fc_b):
    info = plsc.get_sparse_core_info()
    sc_pool = _make_sc_pool(info.num_cores, info.num_subcores)
    pooled_sum = sc_pool(x.astype(jnp.int32), emb_table)
    return _linear(pooled_sum, fc_w, fc_b.reshape(1, OUT))
